# Initial kernel scaffold; baseline (speedup 1.0000x reference)
import jax
import jax.numpy as jnp
from jax.experimental import pallas as pl


def kernel(x, edge_index, W_self, W_inter):
    raise NotImplementedError("write your pallas kernel here")


# speedup vs baseline: 6.7277x; 6.7277x over previous
import functools

import jax
import jax.numpy as jnp
from jax import lax
from jax.experimental import pallas as pl
from jax.experimental.pallas import tpu as pltpu
from jax.experimental.pallas import tpu_sc as plsc

NTILES = 16
NCORES = 2
CH = 128
DEGW = 128


def _row_split(n: int):
    rpt = ((n + NTILES - 1) // NTILES + 7) // 8 * 8
    nft = n // rpt
    last = n - nft * rpt
    return rpt, nft, last


def _degrees_kernel(n: int, e: int):
    nchunks = e // CH
    iters = pl.cdiv(nchunks, NTILES)
    rpt, nft, last = _row_split(n)
    mesh = plsc.VectorSubcoreMesh(core_axis_name="c", subcore_axis_name="s")

    @functools.partial(
        pl.kernel,
        out_type=jax.ShapeDtypeStruct((NCORES, n, DEGW), jnp.float32),
        mesh=mesh,
        scratch_types=[
            pltpu.VMEM((1, CH), jnp.int32),
            pltpu.VMEM((CH, DEGW), jnp.float32),
            pltpu.VMEM((CH, DEGW), jnp.float32),
            pltpu.VMEM_SHARED((n, DEGW), jnp.float32),
        ],
    )
    def k(eidx_hbm, out_hbm, idx_v, ones_v, zero_v, deg_sh):
        c = lax.axis_index("c")
        s = lax.axis_index("s")

        @pl.loop(0, CH)
        def _fill(i):
            ones_v[i, :] = jnp.ones((DEGW,), jnp.float32)
            zero_v[i, :] = jnp.zeros((DEGW,), jnp.float32)

        base = s * rpt

        def _zero(cnt):
            nfull, rem = cnt // CH, cnt % CH

            @pl.loop(0, nfull)
            def _z(j):
                pltpu.sync_copy(zero_v,
                                deg_sh.at[pl.ds(base + j * CH, CH), :])

            if rem:
                pltpu.sync_copy(zero_v.at[pl.ds(0, rem), :],
                                deg_sh.at[pl.ds(base + nfull * CH, rem), :])

        pl.when(s < nft)(lambda: _zero(rpt))
        if last:
            pl.when(s == nft)(lambda: _zero(last))
        plsc.subcore_barrier()

        @pl.loop(0, iters)
        def _chunks(i):
            chunk = i * NTILES + s

            @pl.when(chunk < nchunks)
            def _():
                pltpu.sync_copy(eidx_hbm.at[pl.ds(c * e + chunk * CH, CH)],
                                idx_v.at[0])
                pltpu.sync_copy(ones_v, deg_sh.at[idx_v.at[0]], add=True)

        plsc.subcore_barrier()

        def _drain(cnt):
            pltpu.sync_copy(deg_sh.at[pl.ds(base, cnt), :],
                            out_hbm.at[c, pl.ds(base, cnt), :])

        pl.when(s < nft)(lambda: _drain(rpt))
        if last:
            pl.when(s == nft)(lambda: _drain(last))

    return k


def _aggregate_kernel(n: int, e: int, d: int):
    half = d // 2
    nchunks = e // CH
    iters = pl.cdiv(nchunks, NTILES)
    rpt, nft, last = _row_split(n)
    mesh = plsc.VectorSubcoreMesh(core_axis_name="c", subcore_axis_name="s")

    @functools.partial(
        pl.kernel,
        out_type=jax.ShapeDtypeStruct((n, d), jnp.float32),
        mesh=mesh,
        scratch_types=[
            pltpu.VMEM((CH,), jnp.int32),
            pltpu.VMEM((CH,), jnp.int32),
            pltpu.VMEM((1, CH), jnp.int32),
            pltpu.VMEM((CH, half), jnp.float32),
            pltpu.VMEM_SHARED((n, half), jnp.float32),
        ],
    )
    def k(h2_hbm, src_hbm, dst_hbm, out_hbm, src_v, gidx_v, dst_v, rows_v,
          agg_sh):
        c = lax.axis_index("c")
        s = lax.axis_index("s")

        @pl.loop(0, CH)
        def _zrows(i):
            for j in range(half // 16):
                rows_v[i, pl.ds(j * 16, 16)] = jnp.zeros((16,), jnp.float32)

        base = s * rpt

        def _zero(cnt):
            nfull, rem = cnt // CH, cnt % CH

            @pl.loop(0, nfull)
            def _z(j):
                pltpu.sync_copy(rows_v,
                                agg_sh.at[pl.ds(base + j * CH, CH), :])

            if rem:
                pltpu.sync_copy(rows_v.at[pl.ds(0, rem), :],
                                agg_sh.at[pl.ds(base + nfull * CH, rem), :])

        pl.when(s < nft)(lambda: _zero(rpt))
        if last:
            pl.when(s == nft)(lambda: _zero(last))
        plsc.subcore_barrier()

        @pl.loop(0, iters)
        def _chunks(i):
            chunk = i * NTILES + s

            @pl.when(chunk < nchunks)
            def _():
                pltpu.sync_copy(src_hbm.at[pl.ds(chunk * CH, CH)], src_v)
                pltpu.sync_copy(dst_hbm.at[pl.ds(chunk * CH, CH)],
                                dst_v.at[0])
                for j in range(CH // 16):
                    v = src_v[pl.ds(j * 16, 16)]
                    gidx_v[pl.ds(j * 16, 16)] = v + v + c
                pltpu.sync_copy(h2_hbm.at[gidx_v], rows_v)
                pltpu.sync_copy(rows_v, agg_sh.at[dst_v.at[0]], add=True)

        plsc.subcore_barrier()

        def _drain(cnt):
            pltpu.sync_copy(agg_sh.at[pl.ds(base, cnt), :],
                            out_hbm.at[pl.ds(base, cnt),
                                       pl.ds(c * half, half)])

        pl.when(s < nft)(lambda: _drain(rpt))
        if last:
            pl.when(s == nft)(lambda: _drain(last))

    return k


def _scale_kernel(n: int, d: int, bn: int):
    def body(x_ref, deg_ref, o_ref):
        norm = lax.rsqrt(jnp.maximum(deg_ref[...], 1.0))
        o_ref[...] = x_ref[...] * norm

    return pl.pallas_call(
        body,
        grid=(n // bn,),
        in_specs=[
            pl.BlockSpec((bn, d), lambda i: (i, 0)),
            pl.BlockSpec((bn, 1), lambda i: (i, 0)),
        ],
        out_specs=pl.BlockSpec((bn, d), lambda i: (i, 0)),
        out_shape=jax.ShapeDtypeStruct((n, d), jnp.float32),
    )


def _final_kernel(n: int, d: int, bn: int):
    def body(x_ref, agg_ref, deg_ref, ws_ref, wi_ref, o_ref):
        ni = lax.rsqrt(jnp.maximum(deg_ref[...], 1.0))
        a = agg_ref[...] * ni
        x = x_ref[...]
        o_ref[...] = (
            jnp.dot(a + x, ws_ref[...], preferred_element_type=jnp.float32)
            + jnp.dot(x * a, wi_ref[...], preferred_element_type=jnp.float32)
        )

    return pl.pallas_call(
        body,
        grid=(n // bn,),
        in_specs=[
            pl.BlockSpec((bn, d), lambda i: (i, 0)),
            pl.BlockSpec((bn, d), lambda i: (i, 0)),
            pl.BlockSpec((bn, 1), lambda i: (i, 0)),
            pl.BlockSpec((d, d), lambda i: (0, 0)),
            pl.BlockSpec((d, d), lambda i: (0, 0)),
        ],
        out_specs=pl.BlockSpec((bn, d), lambda i: (i, 0)),
        out_shape=jax.ShapeDtypeStruct((n, d), jnp.float32),
    )


def kernel(x, edge_index, W_self, W_inter):
    n, d = x.shape
    e = edge_index.shape[1]
    assert e % CH == 0 and n % NTILES == 0 and d % 32 == 0

    eidx = edge_index.astype(jnp.int32)
    eidx_flat = eidx.reshape(2 * e)

    degs = _degrees_kernel(n, e)(eidx_flat)
    outdeg = degs[0, :, 0:1]
    indeg = degs[1, :, 0:1]

    h = _scale_kernel(n, d, 1000)(x, outdeg)
    h2 = h.reshape(2 * n, d // 2)

    agg = _aggregate_kernel(n, e, d)(h2, eidx[0], eidx[1])

    return _final_kernel(n, d, 1000)(x, agg, indeg, W_self, W_inter)


# speedup vs baseline: 10.0668x; 1.4963x over previous
import functools

import jax
import jax.numpy as jnp
from jax import lax
from jax.experimental import pallas as pl
from jax.experimental.pallas import tpu as pltpu
from jax.experimental.pallas import tpu_sc as plsc

NTILES = 16
NCORES = 2
CH = 128
DEGW = 128


def _row_split(n: int):
    rpt = ((n + NTILES - 1) // NTILES + 7) // 8 * 8
    nft = n // rpt
    last = n - nft * rpt
    return rpt, nft, last


def _degrees_kernel(n: int, e: int):
    nchunks = e // CH
    iters = pl.cdiv(nchunks, NTILES)
    rpt, nft, last = _row_split(n)
    mesh = plsc.VectorSubcoreMesh(core_axis_name="c", subcore_axis_name="s")

    @functools.partial(
        pl.kernel,
        out_type=jax.ShapeDtypeStruct((NCORES, n, DEGW), jnp.float32),
        mesh=mesh,
        scratch_types=[
            pltpu.VMEM((1, CH), jnp.int32),
            pltpu.VMEM((CH, DEGW), jnp.float32),
            pltpu.VMEM((CH, DEGW), jnp.float32),
            pltpu.VMEM_SHARED((n, DEGW), jnp.float32),
        ],
    )
    def k(eidx_hbm, out_hbm, idx_v, ones_v, zero_v, deg_sh):
        c = lax.axis_index("c")
        s = lax.axis_index("s")

        @pl.loop(0, CH)
        def _fill(i):
            ones_v[i, :] = jnp.ones((DEGW,), jnp.float32)
            zero_v[i, :] = jnp.zeros((DEGW,), jnp.float32)

        base = s * rpt

        def _zero(cnt):
            nfull, rem = cnt // CH, cnt % CH

            @pl.loop(0, nfull)
            def _z(j):
                pltpu.sync_copy(zero_v,
                                deg_sh.at[pl.ds(base + j * CH, CH), :])

            if rem:
                pltpu.sync_copy(zero_v.at[pl.ds(0, rem), :],
                                deg_sh.at[pl.ds(base + nfull * CH, rem), :])

        pl.when(s < nft)(lambda: _zero(rpt))
        if last:
            pl.when(s == nft)(lambda: _zero(last))
        plsc.subcore_barrier()

        @pl.loop(0, iters)
        def _chunks(i):
            chunk = i * NTILES + s

            @pl.when(chunk < nchunks)
            def _():
                pltpu.sync_copy(eidx_hbm.at[pl.ds(c * e + chunk * CH, CH)],
                                idx_v.at[0])
                pltpu.sync_copy(ones_v, deg_sh.at[idx_v.at[0]], add=True)

        plsc.subcore_barrier()

        def _drain(cnt):
            pltpu.sync_copy(deg_sh.at[pl.ds(base, cnt), :],
                            out_hbm.at[c, pl.ds(base, cnt), :])

        pl.when(s < nft)(lambda: _drain(rpt))
        if last:
            pl.when(s == nft)(lambda: _drain(last))

    return k


def _aggregate_kernel(n: int, e: int, d: int):
    half = d // 2
    nchunks = e // CH
    cpt = (pl.cdiv(nchunks, NTILES) + 7) // 8 * 8
    cft = nchunks // cpt
    crem = nchunks - cft * cpt
    NBUF = 2
    PASSES = 2
    sp = cpt // PASSES
    assert cpt % (PASSES * NBUF) == 0
    rpt, nft, last = _row_split(n)
    mesh = plsc.VectorSubcoreMesh(core_axis_name="c", subcore_axis_name="s")

    @functools.partial(
        pl.kernel,
        out_type=jax.ShapeDtypeStruct((n, d), jnp.float32),
        mesh=mesh,
        scratch_types=[
            pltpu.VMEM((sp, CH), jnp.int32),
            pltpu.VMEM((sp, CH), jnp.int32),
            [pltpu.VMEM((CH, half), jnp.float32) for _ in range(NBUF)],
            [pltpu.SemaphoreType.DMA for _ in range(NBUF)],
            [pltpu.SemaphoreType.DMA for _ in range(NBUF)],
            pltpu.VMEM_SHARED((n, half), jnp.float32),
        ],
    )
    def k(h2_hbm, src_hbm, dst_hbm, out_hbm, gidx_v, dst_v, rows,
          gsem, ssem, agg_sh):
        c = lax.axis_index("c")
        s = lax.axis_index("s")

        nloc = jnp.where(s < cft, cpt, jnp.where(s == cft, crem, 0))
        cb = s * cpt

        @pl.loop(0, CH)
        def _zrows(i):
            for jj in range(half // 16):
                rows[0][i, pl.ds(jj * 16, 16)] = jnp.zeros((16,), jnp.float32)

        base = s * rpt

        def _zero(cnt):
            nfull, rem = cnt // CH, cnt % CH

            @pl.loop(0, nfull)
            def _z(j):
                pltpu.sync_copy(rows[0],
                                agg_sh.at[pl.ds(base + j * CH, CH), :])

            if rem:
                pltpu.sync_copy(rows[0].at[pl.ds(0, rem), :],
                                agg_sh.at[pl.ds(base + nfull * CH, rem), :])

        pl.when(s < nft)(lambda: _zero(rpt))
        if last:
            pl.when(s == nft)(lambda: _zero(last))
        plsc.subcore_barrier()

        def _gather(j, b):
            pltpu.async_copy(h2_hbm.at[gidx_v.at[j]], rows[b], gsem[b])

        def _scat_desc(j, b):
            return pltpu.make_async_copy(
                rows[b], agg_sh.at[dst_v.at[j]], ssem[b])

        for p in range(PASSES):
            np_ = jnp.clip(nloc - p * sp, 0, sp)

            pltpu.sync_copy(src_hbm.at[pl.ds(cb + p * sp, sp), :], gidx_v)
            pltpu.sync_copy(dst_hbm.at[pl.ds(cb + p * sp, sp), :], dst_v)

            @pl.loop(0, sp)
            def _gx(i):
                for jj in range(CH // 16):
                    v = gidx_v[i, pl.ds(jj * 16, 16)]
                    gidx_v[i, pl.ds(jj * 16, 16)] = v + v + c

            for b in range(NBUF):
                pl.when(b < np_)(functools.partial(_gather, b, b))

            @pl.loop(0, sp // NBUF)
            def _steady(o):
                for b in range(NBUF):
                    j = o * NBUF + b

                    @pl.when(j < np_)
                    def _():
                        pltpu.make_async_copy(h2_hbm.at[gidx_v.at[j]],
                                              rows[b], gsem[b]).wait()
                        pltpu.async_copy(rows[b], agg_sh.at[dst_v.at[j]],
                                         ssem[b], add=True)

                    nj = j + NBUF

                    @pl.when(nj < np_)
                    def _():
                        _scat_desc(j, b).wait()
                        _gather(nj, b)

            for b in range(NBUF):
                @pl.when(jnp.maximum(np_ - NBUF, 0) + b < np_)
                def _():
                    _scat_desc(0, b).wait()

        plsc.subcore_barrier()

        def _drain(cnt):
            pltpu.sync_copy(agg_sh.at[pl.ds(base, cnt), :],
                            out_hbm.at[pl.ds(base, cnt),
                                       pl.ds(c * half, half)])

        pl.when(s < nft)(lambda: _drain(rpt))
        if last:
            pl.when(s == nft)(lambda: _drain(last))

    return k


def _scale_kernel(n: int, d: int, bn: int):
    def body(x_ref, deg_ref, o_ref):
        norm = lax.rsqrt(jnp.maximum(deg_ref[:, 0:1], 1.0))
        o_ref[...] = x_ref[...] * norm

    return pl.pallas_call(
        body,
        grid=(n // bn,),
        in_specs=[
            pl.BlockSpec((bn, d), lambda i: (i, 0)),
            pl.BlockSpec((bn, DEGW), lambda i: (i, 0)),
        ],
        out_specs=pl.BlockSpec((bn, d), lambda i: (i, 0)),
        out_shape=jax.ShapeDtypeStruct((n, d), jnp.float32),
    )


def _final_kernel(n: int, d: int, bn: int):
    def body(x_ref, agg_ref, deg_ref, ws_ref, wi_ref, o_ref):
        ni = lax.rsqrt(jnp.maximum(deg_ref[:, 0:1], 1.0))
        a = agg_ref[...] * ni
        x = x_ref[...]
        o_ref[...] = (
            jnp.dot(a + x, ws_ref[...], preferred_element_type=jnp.float32)
            + jnp.dot(x * a, wi_ref[...], preferred_element_type=jnp.float32)
        )

    return pl.pallas_call(
        body,
        grid=(n // bn,),
        in_specs=[
            pl.BlockSpec((bn, d), lambda i: (i, 0)),
            pl.BlockSpec((bn, d), lambda i: (i, 0)),
            pl.BlockSpec((bn, DEGW), lambda i: (i, 0)),
            pl.BlockSpec((d, d), lambda i: (0, 0)),
            pl.BlockSpec((d, d), lambda i: (0, 0)),
        ],
        out_specs=pl.BlockSpec((bn, d), lambda i: (i, 0)),
        out_shape=jax.ShapeDtypeStruct((n, d), jnp.float32),
    )


def kernel(x, edge_index, W_self, W_inter):
    n, d = x.shape
    e = edge_index.shape[1]
    assert e % CH == 0 and n % NTILES == 0 and d % 32 == 0

    eidx = edge_index.astype(jnp.int32)
    eidx_flat = eidx.reshape(2 * e)

    degs = _degrees_kernel(n, e)(eidx_flat)

    h = _scale_kernel(n, d, 1000)(x, degs[0])
    h2 = h.reshape(2 * n, d // 2)

    nchunks = e // CH
    cpt = (-(-nchunks // NTILES) + 7) // 8 * 8
    pad_rows = NTILES * cpt - nchunks
    src2d = eidx[0].reshape(nchunks, CH)
    dst2d = eidx[1].reshape(nchunks, CH)
    if pad_rows:
        zpad = jnp.zeros((pad_rows, CH), jnp.int32)
        src2d = jnp.concatenate([src2d, zpad], axis=0)
        dst2d = jnp.concatenate([dst2d, zpad], axis=0)
    agg = _aggregate_kernel(n, e, d)(h2, src2d, dst2d)

    return _final_kernel(n, d, 1000)(x, agg, degs[1], W_self, W_inter)


# speedup vs baseline: 10.9755x; 1.0903x over previous
import functools

import jax
import jax.numpy as jnp
from jax import lax
from jax.experimental import pallas as pl
from jax.experimental.pallas import tpu as pltpu
from jax.experimental.pallas import tpu_sc as plsc

NTILES = 16
NCORES = 2
CH = 128
DEGW = 128


def _row_split(n: int):
    rpt = ((n + NTILES - 1) // NTILES + 7) // 8 * 8
    nft = n // rpt
    last = n - nft * rpt
    return rpt, nft, last


def _degrees_kernel(n: int, e: int):
    nchunks = e // CH
    cpt = (pl.cdiv(nchunks, NTILES) + 7) // 8 * 8
    cft = nchunks // cpt
    crem = nchunks - cft * cpt
    W = 8
    rpt, nft, last = _row_split(n)
    mesh = plsc.VectorSubcoreMesh(core_axis_name="c", subcore_axis_name="s")

    @functools.partial(
        pl.kernel,
        out_type=jax.ShapeDtypeStruct((NCORES, n, DEGW), jnp.float32),
        mesh=mesh,
        scratch_types=[
            pltpu.VMEM((cpt, CH), jnp.int32),
            pltpu.VMEM((CH, DEGW), jnp.float32),
            pltpu.VMEM((CH, DEGW), jnp.float32),
            pltpu.SemaphoreType.DMA,
            pltpu.VMEM_SHARED((n, DEGW), jnp.float32),
        ],
    )
    def k(eidx_hbm, out_hbm, idx_v, ones_v, zero_v, sem, deg_sh):
        c = lax.axis_index("c")
        s = lax.axis_index("s")

        nloc = jnp.where(s < cft, cpt, jnp.where(s == cft, crem, 0))
        pltpu.sync_copy(eidx_hbm.at[pl.ds((c * NTILES + s) * cpt, cpt), :],
                        idx_v)

        @pl.loop(0, CH)
        def _fill(i):
            ones_v[i, :] = jnp.ones((DEGW,), jnp.float32)
            zero_v[i, :] = jnp.zeros((DEGW,), jnp.float32)

        base = s * rpt

        def _zero(cnt):
            nfull, rem = cnt // CH, cnt % CH

            @pl.loop(0, nfull)
            def _z(j):
                pltpu.sync_copy(zero_v,
                                deg_sh.at[pl.ds(base + j * CH, CH), :])

            if rem:
                pltpu.sync_copy(zero_v.at[pl.ds(0, rem), :],
                                deg_sh.at[pl.ds(base + nfull * CH, rem), :])

        pl.when(s < nft)(lambda: _zero(rpt))
        if last:
            pl.when(s == nft)(lambda: _zero(last))
        plsc.subcore_barrier()

        def _desc(j):
            return pltpu.make_async_copy(ones_v, deg_sh.at[idx_v.at[j]], sem)

        @pl.loop(0, cpt)
        def _chunks(j):
            @pl.when(j < nloc)
            def _():
                pltpu.async_copy(ones_v, deg_sh.at[idx_v.at[j]], sem,
                                 add=True)

            @pl.when((j >= W) & (j < nloc))
            def _():
                _desc(0).wait()

        for w in range(W):
            @pl.when(jnp.maximum(nloc - W, 0) + w < nloc)
            def _():
                _desc(0).wait()

        plsc.subcore_barrier()

        def _drain(cnt):
            pltpu.sync_copy(deg_sh.at[pl.ds(base, cnt), :],
                            out_hbm.at[c, pl.ds(base, cnt), :])

        pl.when(s < nft)(lambda: _drain(rpt))
        if last:
            pl.when(s == nft)(lambda: _drain(last))

    return k


def _aggregate_kernel(n: int, e: int, d: int):
    half = d // 2
    nchunks = e // CH
    cpt = (pl.cdiv(nchunks, NTILES) + 7) // 8 * 8
    cft = nchunks // cpt
    crem = nchunks - cft * cpt
    NBUF = 2
    PASSES = 2
    sp = cpt // PASSES
    assert cpt % (PASSES * NBUF) == 0
    rpt, nft, last = _row_split(n)
    mesh = plsc.VectorSubcoreMesh(core_axis_name="c", subcore_axis_name="s")

    @functools.partial(
        pl.kernel,
        out_type=jax.ShapeDtypeStruct((n, d), jnp.float32),
        mesh=mesh,
        scratch_types=[
            pltpu.VMEM((sp, CH), jnp.int32),
            pltpu.VMEM((sp, CH), jnp.int32),
            [pltpu.VMEM((CH, half), jnp.float32) for _ in range(NBUF)],
            [pltpu.SemaphoreType.DMA for _ in range(NBUF)],
            [pltpu.SemaphoreType.DMA for _ in range(NBUF)],
            pltpu.VMEM_SHARED((n, half), jnp.float32),
        ],
    )
    def k(h2_hbm, src_hbm, dst_hbm, out_hbm, gidx_v, dst_v, rows,
          gsem, ssem, agg_sh):
        c = lax.axis_index("c")
        s = lax.axis_index("s")

        nloc = jnp.where(s < cft, cpt, jnp.where(s == cft, crem, 0))
        cb = s * cpt

        @pl.loop(0, CH)
        def _zrows(i):
            for jj in range(half // 16):
                rows[0][i, pl.ds(jj * 16, 16)] = jnp.zeros((16,), jnp.float32)

        base = s * rpt

        def _zero(cnt):
            nfull, rem = cnt // CH, cnt % CH

            @pl.loop(0, nfull)
            def _z(j):
                pltpu.sync_copy(rows[0],
                                agg_sh.at[pl.ds(base + j * CH, CH), :])

            if rem:
                pltpu.sync_copy(rows[0].at[pl.ds(0, rem), :],
                                agg_sh.at[pl.ds(base + nfull * CH, rem), :])

        pl.when(s < nft)(lambda: _zero(rpt))
        if last:
            pl.when(s == nft)(lambda: _zero(last))
        plsc.subcore_barrier()

        def _gather(j, b):
            pltpu.async_copy(h2_hbm.at[gidx_v.at[j]], rows[b], gsem[b])

        def _scat_desc(j, b):
            return pltpu.make_async_copy(
                rows[b], agg_sh.at[dst_v.at[j]], ssem[b])

        for p in range(PASSES):
            np_ = jnp.clip(nloc - p * sp, 0, sp)

            pltpu.sync_copy(src_hbm.at[pl.ds(cb + p * sp, sp), :], gidx_v)
            pltpu.sync_copy(dst_hbm.at[pl.ds(cb + p * sp, sp), :], dst_v)

            @pl.loop(0, sp)
            def _gx(i):
                for jj in range(CH // 16):
                    v = gidx_v[i, pl.ds(jj * 16, 16)]
                    gidx_v[i, pl.ds(jj * 16, 16)] = v + v + c

            for b in range(NBUF):
                pl.when(b < np_)(functools.partial(_gather, b, b))

            @pl.loop(0, sp // NBUF)
            def _steady(o):
                for b in range(NBUF):
                    j = o * NBUF + b

                    @pl.when(j < np_)
                    def _():
                        pltpu.make_async_copy(h2_hbm.at[gidx_v.at[j]],
                                              rows[b], gsem[b]).wait()
                        pltpu.async_copy(rows[b], agg_sh.at[dst_v.at[j]],
                                         ssem[b], add=True)

                    nj = j + NBUF

                    @pl.when(nj < np_)
                    def _():
                        _scat_desc(j, b).wait()
                        _gather(nj, b)

            for b in range(NBUF):
                @pl.when(jnp.maximum(np_ - NBUF, 0) + b < np_)
                def _():
                    _scat_desc(0, b).wait()

        plsc.subcore_barrier()

        def _drain(cnt):
            pltpu.sync_copy(agg_sh.at[pl.ds(base, cnt), :],
                            out_hbm.at[pl.ds(base, cnt),
                                       pl.ds(c * half, half)])

        pl.when(s < nft)(lambda: _drain(rpt))
        if last:
            pl.when(s == nft)(lambda: _drain(last))

    return k


def _scale_kernel(n: int, d: int, bn: int):
    def body(x_ref, deg_ref, o_ref):
        norm = lax.rsqrt(jnp.maximum(deg_ref[...], 1.0))
        o_ref[...] = x_ref[...] * norm

    return pl.pallas_call(
        body,
        grid=(n // bn,),
        in_specs=[
            pl.BlockSpec((bn, d), lambda i: (i, 0)),
            pl.BlockSpec((bn, 1), lambda i: (i, 0)),
        ],
        out_specs=pl.BlockSpec((bn, d), lambda i: (i, 0)),
        out_shape=jax.ShapeDtypeStruct((n, d), jnp.float32),
    )


def _final_kernel(n: int, d: int, bn: int):
    def body(x_ref, agg_ref, deg_ref, ws_ref, wi_ref, o_ref):
        ni = lax.rsqrt(jnp.maximum(deg_ref[...], 1.0))
        a = agg_ref[...] * ni
        x = x_ref[...]
        o_ref[...] = (
            jnp.dot(a + x, ws_ref[...], preferred_element_type=jnp.float32)
            + jnp.dot(x * a, wi_ref[...], preferred_element_type=jnp.float32)
        )

    return pl.pallas_call(
        body,
        grid=(n // bn,),
        in_specs=[
            pl.BlockSpec((bn, d), lambda i: (i, 0)),
            pl.BlockSpec((bn, d), lambda i: (i, 0)),
            pl.BlockSpec((bn, 1), lambda i: (i, 0)),
            pl.BlockSpec((d, d), lambda i: (0, 0)),
            pl.BlockSpec((d, d), lambda i: (0, 0)),
        ],
        out_specs=pl.BlockSpec((bn, d), lambda i: (i, 0)),
        out_shape=jax.ShapeDtypeStruct((n, d), jnp.float32),
    )


def kernel(x, edge_index, W_self, W_inter):
    n, d = x.shape
    e = edge_index.shape[1]
    assert e % CH == 0 and n % NTILES == 0 and d % 32 == 0

    eidx = edge_index.astype(jnp.int32)
    nchunks = e // CH
    cpt = (-(-nchunks // NTILES) + 7) // 8 * 8
    pad_rows = NTILES * cpt - nchunks
    src2d = eidx[0].reshape(nchunks, CH)
    dst2d = eidx[1].reshape(nchunks, CH)
    if pad_rows:
        zpad = jnp.zeros((pad_rows, CH), jnp.int32)
        src2d = jnp.concatenate([src2d, zpad], axis=0)
        dst2d = jnp.concatenate([dst2d, zpad], axis=0)

    degs = _degrees_kernel(n, e)(jnp.concatenate([src2d, dst2d], axis=0))

    h = _scale_kernel(n, d, 1000)(x, degs[0, :, 0:1])
    h2 = h.reshape(2 * n, d // 2)

    agg = _aggregate_kernel(n, e, d)(h2, src2d, dst2d)

    return _final_kernel(n, d, 1000)(x, agg, degs[1, :, 0:1], W_self,
                                     W_inter)


# speedup vs baseline: 10.9810x; 1.0005x over previous
import functools

import jax
import jax.numpy as jnp
from jax import lax
from jax.experimental import pallas as pl
from jax.experimental.pallas import tpu as pltpu
from jax.experimental.pallas import tpu_sc as plsc

NTILES = 16
NCORES = 2
CH = 128
DEGW = 128


def _row_split(n: int):
    rpt = ((n + NTILES - 1) // NTILES + 7) // 8 * 8
    nft = n // rpt
    last = n - nft * rpt
    return rpt, nft, last


def _degrees_kernel(n: int, e: int):
    nchunks = e // CH
    cpt = (pl.cdiv(nchunks, NTILES) + 7) // 8 * 8
    cft = nchunks // cpt
    crem = nchunks - cft * cpt
    W = 16
    rpt, nft, last = _row_split(n)
    mesh = plsc.VectorSubcoreMesh(core_axis_name="c", subcore_axis_name="s")

    @functools.partial(
        pl.kernel,
        out_type=jax.ShapeDtypeStruct((NCORES, n, DEGW), jnp.float32),
        mesh=mesh,
        scratch_types=[
            pltpu.VMEM((cpt, CH), jnp.int32),
            pltpu.VMEM((CH, DEGW), jnp.float32),
            pltpu.VMEM((CH, DEGW), jnp.float32),
            pltpu.SemaphoreType.DMA,
            pltpu.VMEM_SHARED((n, DEGW), jnp.float32),
        ],
    )
    def k(eidx_hbm, out_hbm, idx_v, ones_v, zero_v, sem, deg_sh):
        c = lax.axis_index("c")
        s = lax.axis_index("s")

        nloc = jnp.where(s < cft, cpt, jnp.where(s == cft, crem, 0))
        pltpu.sync_copy(eidx_hbm.at[pl.ds((c * NTILES + s) * cpt, cpt), :],
                        idx_v)

        @pl.loop(0, CH)
        def _fill(i):
            ones_v[i, :] = jnp.ones((DEGW,), jnp.float32)
            zero_v[i, :] = jnp.zeros((DEGW,), jnp.float32)

        base = s * rpt

        def _zero(cnt):
            nfull, rem = cnt // CH, cnt % CH

            @pl.loop(0, nfull)
            def _z(j):
                pltpu.sync_copy(zero_v,
                                deg_sh.at[pl.ds(base + j * CH, CH), :])

            if rem:
                pltpu.sync_copy(zero_v.at[pl.ds(0, rem), :],
                                deg_sh.at[pl.ds(base + nfull * CH, rem), :])

        pl.when(s < nft)(lambda: _zero(rpt))
        if last:
            pl.when(s == nft)(lambda: _zero(last))
        plsc.subcore_barrier()

        def _desc(j):
            return pltpu.make_async_copy(ones_v, deg_sh.at[idx_v.at[j]], sem)

        @pl.loop(0, cpt)
        def _chunks(j):
            @pl.when(j < nloc)
            def _():
                pltpu.async_copy(ones_v, deg_sh.at[idx_v.at[j]], sem,
                                 add=True)

            @pl.when((j >= W) & (j < nloc))
            def _():
                _desc(0).wait()

        for w in range(W):
            @pl.when(jnp.maximum(nloc - W, 0) + w < nloc)
            def _():
                _desc(0).wait()

        plsc.subcore_barrier()

        def _drain(cnt):
            pltpu.sync_copy(deg_sh.at[pl.ds(base, cnt), :],
                            out_hbm.at[c, pl.ds(base, cnt), :])

        pl.when(s < nft)(lambda: _drain(rpt))
        if last:
            pl.when(s == nft)(lambda: _drain(last))

    return k


def _aggregate_kernel(n: int, e: int, d: int):
    half = d // 2
    nchunks = e // CH
    cpt = (pl.cdiv(nchunks, NTILES) + 7) // 8 * 8
    cft = nchunks // cpt
    crem = nchunks - cft * cpt
    NBUF = 2
    PASSES = 2
    sp = cpt // PASSES
    assert cpt % (PASSES * NBUF) == 0
    rpt, nft, last = _row_split(n)
    mesh = plsc.VectorSubcoreMesh(core_axis_name="c", subcore_axis_name="s")

    @functools.partial(
        pl.kernel,
        out_type=jax.ShapeDtypeStruct((n, d), jnp.float32),
        mesh=mesh,
        scratch_types=[
            pltpu.VMEM((sp, CH), jnp.int32),
            pltpu.VMEM((sp, CH), jnp.int32),
            [pltpu.VMEM((CH, half), jnp.float32) for _ in range(NBUF)],
            [pltpu.SemaphoreType.DMA for _ in range(NBUF)],
            [pltpu.SemaphoreType.DMA for _ in range(NBUF)],
            pltpu.VMEM_SHARED((n, half), jnp.float32),
        ],
    )
    def k(h2_hbm, src_hbm, dst_hbm, out_hbm, gidx_v, dst_v, rows,
          gsem, ssem, agg_sh):
        c = lax.axis_index("c")
        s = lax.axis_index("s")

        nloc = jnp.where(s < cft, cpt, jnp.where(s == cft, crem, 0))
        cb = s * cpt

        @pl.loop(0, CH)
        def _zrows(i):
            for jj in range(half // 16):
                rows[0][i, pl.ds(jj * 16, 16)] = jnp.zeros((16,), jnp.float32)

        base = s * rpt

        def _zero(cnt):
            nfull, rem = cnt // CH, cnt % CH

            @pl.loop(0, nfull)
            def _z(j):
                pltpu.sync_copy(rows[0],
                                agg_sh.at[pl.ds(base + j * CH, CH), :])

            if rem:
                pltpu.sync_copy(rows[0].at[pl.ds(0, rem), :],
                                agg_sh.at[pl.ds(base + nfull * CH, rem), :])

        pl.when(s < nft)(lambda: _zero(rpt))
        if last:
            pl.when(s == nft)(lambda: _zero(last))
        plsc.subcore_barrier()

        def _gather(j, b):
            pltpu.async_copy(h2_hbm.at[gidx_v.at[j]], rows[b], gsem[b])

        def _scat_desc(j, b):
            return pltpu.make_async_copy(
                rows[b], agg_sh.at[dst_v.at[j]], ssem[b])

        for p in range(PASSES):
            np_ = jnp.clip(nloc - p * sp, 0, sp)

            pltpu.sync_copy(src_hbm.at[pl.ds(cb + p * sp, sp), :], gidx_v)
            pltpu.sync_copy(dst_hbm.at[pl.ds(cb + p * sp, sp), :], dst_v)

            @pl.loop(0, sp)
            def _gx(i):
                for jj in range(CH // 16):
                    v = gidx_v[i, pl.ds(jj * 16, 16)]
                    gidx_v[i, pl.ds(jj * 16, 16)] = v + v + c

            for b in range(NBUF):
                pl.when(b < np_)(functools.partial(_gather, b, b))

            @pl.loop(0, sp // NBUF)
            def _steady(o):
                for b in range(NBUF):
                    j = o * NBUF + b

                    @pl.when(j < np_)
                    def _():
                        pltpu.make_async_copy(h2_hbm.at[gidx_v.at[j]],
                                              rows[b], gsem[b]).wait()
                        pltpu.async_copy(rows[b], agg_sh.at[dst_v.at[j]],
                                         ssem[b], add=True)

                    nj = j + NBUF

                    @pl.when(nj < np_)
                    def _():
                        _scat_desc(j, b).wait()
                        _gather(nj, b)

            for b in range(NBUF):
                @pl.when(jnp.maximum(np_ - NBUF, 0) + b < np_)
                def _():
                    _scat_desc(0, b).wait()

        plsc.subcore_barrier()

        def _drain(cnt):
            pltpu.sync_copy(agg_sh.at[pl.ds(base, cnt), :],
                            out_hbm.at[pl.ds(base, cnt),
                                       pl.ds(c * half, half)])

        pl.when(s < nft)(lambda: _drain(rpt))
        if last:
            pl.when(s == nft)(lambda: _drain(last))

    return k


def _scale_kernel(n: int, d: int, bn: int):
    def body(x_ref, deg_ref, o_ref):
        norm = lax.rsqrt(jnp.maximum(deg_ref[...], 1.0))
        o_ref[...] = x_ref[...] * norm

    return pl.pallas_call(
        body,
        grid=(n // bn,),
        in_specs=[
            pl.BlockSpec((bn, d), lambda i: (i, 0)),
            pl.BlockSpec((bn, 1), lambda i: (i, 0)),
        ],
        out_specs=pl.BlockSpec((bn, d), lambda i: (i, 0)),
        out_shape=jax.ShapeDtypeStruct((n, d), jnp.float32),
    )


def _final_kernel(n: int, d: int, bn: int):
    def body(x_ref, agg_ref, deg_ref, ws_ref, wi_ref, o_ref):
        ni = lax.rsqrt(jnp.maximum(deg_ref[...], 1.0))
        a = agg_ref[...] * ni
        x = x_ref[...]
        o_ref[...] = (
            jnp.dot(a + x, ws_ref[...], preferred_element_type=jnp.float32)
            + jnp.dot(x * a, wi_ref[...], preferred_element_type=jnp.float32)
        )

    return pl.pallas_call(
        body,
        grid=(n // bn,),
        in_specs=[
            pl.BlockSpec((bn, d), lambda i: (i, 0)),
            pl.BlockSpec((bn, d), lambda i: (i, 0)),
            pl.BlockSpec((bn, 1), lambda i: (i, 0)),
            pl.BlockSpec((d, d), lambda i: (0, 0)),
            pl.BlockSpec((d, d), lambda i: (0, 0)),
        ],
        out_specs=pl.BlockSpec((bn, d), lambda i: (i, 0)),
        out_shape=jax.ShapeDtypeStruct((n, d), jnp.float32),
    )


def kernel(x, edge_index, W_self, W_inter):
    n, d = x.shape
    e = edge_index.shape[1]
    assert e % CH == 0 and n % NTILES == 0 and d % 32 == 0

    eidx = edge_index.astype(jnp.int32)
    nchunks = e // CH
    cpt = (-(-nchunks // NTILES) + 7) // 8 * 8
    pad_rows = NTILES * cpt - nchunks
    src2d = eidx[0].reshape(nchunks, CH)
    dst2d = eidx[1].reshape(nchunks, CH)
    if pad_rows:
        zpad = jnp.zeros((pad_rows, CH), jnp.int32)
        src2d = jnp.concatenate([src2d, zpad], axis=0)
        dst2d = jnp.concatenate([dst2d, zpad], axis=0)

    degs = _degrees_kernel(n, e)(jnp.concatenate([src2d, dst2d], axis=0))

    h = _scale_kernel(n, d, 1000)(x, degs[0, :, 0:1])
    h2 = h.reshape(2 * n, d // 2)

    agg = _aggregate_kernel(n, e, d)(h2, src2d, dst2d)

    return _final_kernel(n, d, 1000)(x, agg, degs[1, :, 0:1], W_self,
                                     W_inter)
